# baseline (speedup 1.0000x reference)
import functools

import jax
import jax.numpy as jnp
from jax.experimental import pallas as pl
from jax.experimental.pallas import tpu as pltpu

_TOP_K = 8
_BLOCK_N = 256
_BLOCK_D = 1024


def _gating_block_kernel(x_ref, w_ref, b_ref, gates_ref, load_ref, acc_ref,
                         *, k_top, nd):
    x = x_ref[...]
    w = w_ref[...]
    e = w.shape[0]
    bn = x.shape[0]
    j = pl.program_id(1)
    k = pl.program_id(2)
    partial = jax.lax.dot_general(
        w, x, (((1,), (1,)), ((), ())), preferred_element_type=jnp.float32)

    col = pl.ds(k * bn, bn)

    @pl.when(j == 0)
    def _first_chunk():
        acc_ref[:, col] = partial + b_ref[...].reshape(e, 1)

    @pl.when(j != 0)
    def _accumulate_chunk():
        acc_ref[:, col] += partial

    @pl.when(j == nd - 1)
    def _epilogue():
        logits = acc_ref[:, col]
        row = jax.lax.broadcasted_iota(jnp.int32, (e, bn), 0)
        work = logits
        m0 = jnp.max(work, axis=0, keepdims=True)
        for t in range(k_top):
            m = m0 if t == 0 else jnp.max(work, axis=0, keepdims=True)
            is_max = work == m
            sel = jnp.min(jnp.where(is_max, row, e), axis=0, keepdims=True)
            work = jnp.where(row == sel, -jnp.inf, work)
        selected = work == -jnp.inf
        ex = jnp.where(selected, jnp.exp(logits - m0), jnp.float32(0.0))
        denom = jnp.sum(ex, axis=0, keepdims=True)
        gates = (ex / denom).T
        gates_ref[...] = gates
        counts = jnp.sum((gates > 0).astype(jnp.int32), axis=0, keepdims=True)

        @pl.when(k == 0)
        def _init():
            load_ref[...] = counts[None]

        @pl.when(k != 0)
        def _acc():
            load_ref[...] += counts[None]


def kernel(x, w_gate, b_gate, w_noise, b_noise):
    del w_noise, b_noise
    n, d = x.shape
    e = w_gate.shape[0]
    bn = min(_BLOCK_N, n)
    nb = n // bn
    cores = 2 if nb % 2 == 0 else 1
    half = nb // cores
    bd = min(_BLOCK_D, d)
    nd = d // bd
    b2 = b_gate.reshape(1, e)

    gates, load3 = pl.pallas_call(
        functools.partial(_gating_block_kernel, k_top=_TOP_K, nd=nd),
        grid=(cores, nd, half),
        in_specs=[
            pl.BlockSpec((bn, bd), lambda i, j, k: (i * half + k, j)),
            pl.BlockSpec((e, bd), lambda i, j, k: (0, j)),
            pl.BlockSpec((1, e), lambda i, j, k: (0, 0)),
        ],
        out_specs=[
            pl.BlockSpec((bn, e), lambda i, j, k: (i * half + k, 0)),
            pl.BlockSpec((1, 1, e), lambda i, j, k: (i, 0, 0)),
        ],
        out_shape=[
            jax.ShapeDtypeStruct((n, e), x.dtype),
            jax.ShapeDtypeStruct((cores, 1, e), jnp.int32),
        ],
        scratch_shapes=[pltpu.VMEM((e, half * bn), jnp.float32)],
        compiler_params=pltpu.CompilerParams(
            dimension_semantics=("parallel", "arbitrary", "arbitrary")),
    )(x, w_gate, b2)

    load = load3.sum(axis=(0, 1))
    return gates, load


# speedup vs baseline: 1.9248x; 1.9248x over previous
import functools

import jax
import jax.numpy as jnp
from jax.experimental import pallas as pl
from jax.experimental.pallas import tpu as pltpu

_TOP_K = 8
_BLOCK_N = 256


def _gating_block_kernel(x_ref, w_ref, b_ref, gates_ref, load_ref, *, k_top):
    x = x_ref[...]
    w = w_ref[...]
    e = w.shape[0]
    bn = x.shape[0]
    logits = jax.lax.dot_general(
        w, x, (((1,), (1,)), ((), ())), preferred_element_type=jnp.float32)
    logits = logits + b_ref[...].reshape(e, 1)

    row = jax.lax.broadcasted_iota(jnp.int32, (e, bn), 0)
    work = logits
    m0 = jnp.max(work, axis=0, keepdims=True)
    for t in range(k_top):
        m = m0 if t == 0 else jnp.max(work, axis=0, keepdims=True)
        is_max = work == m
        sel = jnp.min(jnp.where(is_max, row, e), axis=0, keepdims=True)
        work = jnp.where(row == sel, -jnp.inf, work)
    selected = work == -jnp.inf
    ex = jnp.where(selected, jnp.exp(logits - m0), jnp.float32(0.0))
    denom = jnp.sum(ex, axis=0, keepdims=True)
    gates = (ex / denom).T
    gates_ref[...] = gates
    counts = jnp.sum((gates > 0).astype(jnp.int32), axis=0, keepdims=True)

    @pl.when(pl.program_id(1) == 0)
    def _init():
        load_ref[...] = counts[None]

    @pl.when(pl.program_id(1) != 0)
    def _accumulate():
        load_ref[...] += counts[None]


def kernel(x, w_gate, b_gate, w_noise, b_noise):
    del w_noise, b_noise
    n, d = x.shape
    e = w_gate.shape[0]
    bn = min(_BLOCK_N, n)
    nb = n // bn
    cores = 2 if nb % 2 == 0 else 1
    half = nb // cores
    b2 = b_gate.reshape(1, e)

    gates, load3 = pl.pallas_call(
        functools.partial(_gating_block_kernel, k_top=_TOP_K),
        grid=(cores, half),
        in_specs=[
            pl.BlockSpec((bn, d), lambda i, j: (i * half + j, 0)),
            pl.BlockSpec((e, d), lambda i, j: (0, 0)),
            pl.BlockSpec((1, e), lambda i, j: (0, 0)),
        ],
        out_specs=[
            pl.BlockSpec((bn, e), lambda i, j: (i * half + j, 0)),
            pl.BlockSpec((1, 1, e), lambda i, j: (i, 0, 0)),
        ],
        out_shape=[
            jax.ShapeDtypeStruct((n, e), x.dtype),
            jax.ShapeDtypeStruct((cores, 1, e), jnp.int32),
        ],
        compiler_params=pltpu.CompilerParams(
            dimension_semantics=("parallel", "arbitrary")),
    )(x, w_gate, b2)

    load = load3.sum(axis=(0, 1))
    return gates, load


# speedup vs baseline: 2.3033x; 1.1966x over previous
import functools

import jax
import jax.numpy as jnp
from jax.experimental import pallas as pl
from jax.experimental.pallas import tpu as pltpu

_TOP_K = 8
_BLOCK_N = 512


def _gating_block_kernel(x_ref, w_ref, b_ref, gates_ref, load_ref, *, k_top):
    x = x_ref[...]
    w = w_ref[...]
    e = w.shape[0]
    bn = x.shape[0]
    logits = jax.lax.dot_general(
        w, x, (((1,), (1,)), ((), ())), preferred_element_type=jnp.float32)
    logits = logits + b_ref[...].reshape(e, 1)

    row = jax.lax.broadcasted_iota(jnp.int32, (e, bn), 0)
    work = logits
    m0 = jnp.max(work, axis=0, keepdims=True)
    for t in range(k_top):
        m = m0 if t == 0 else jnp.max(work, axis=0, keepdims=True)
        is_max = work == m
        sel = jnp.min(jnp.where(is_max, row, e), axis=0, keepdims=True)
        work = jnp.where(row == sel, -jnp.inf, work)
    selected = work == -jnp.inf
    ex = jnp.where(selected, jnp.exp(logits - m0), jnp.float32(0.0))
    denom = jnp.sum(ex, axis=0, keepdims=True)
    gates = (ex / denom).T
    gates_ref[...] = gates
    counts = jnp.sum((gates > 0).astype(jnp.int32), axis=0, keepdims=True)

    @pl.when(pl.program_id(1) == 0)
    def _init():
        load_ref[...] = counts[None]

    @pl.when(pl.program_id(1) != 0)
    def _accumulate():
        load_ref[...] += counts[None]


def kernel(x, w_gate, b_gate, w_noise, b_noise):
    del w_noise, b_noise
    n, d = x.shape
    e = w_gate.shape[0]
    bn = min(_BLOCK_N, n)
    nb = n // bn
    cores = 2 if nb % 2 == 0 else 1
    half = nb // cores
    b2 = b_gate.reshape(1, e)

    gates, load3 = pl.pallas_call(
        functools.partial(_gating_block_kernel, k_top=_TOP_K),
        grid=(cores, half),
        in_specs=[
            pl.BlockSpec((bn, d), lambda i, j: (i * half + j, 0)),
            pl.BlockSpec((e, d), lambda i, j: (0, 0)),
            pl.BlockSpec((1, e), lambda i, j: (0, 0)),
        ],
        out_specs=[
            pl.BlockSpec((bn, e), lambda i, j: (i * half + j, 0)),
            pl.BlockSpec((1, 1, e), lambda i, j: (i, 0, 0)),
        ],
        out_shape=[
            jax.ShapeDtypeStruct((n, e), x.dtype),
            jax.ShapeDtypeStruct((cores, 1, e), jnp.int32),
        ],
        compiler_params=pltpu.CompilerParams(
            dimension_semantics=("parallel", "arbitrary")),
    )(x, w_gate, b2)

    load = load3.sum(axis=(0, 1))
    return gates, load


# speedup vs baseline: 2.4132x; 1.0477x over previous
import functools

import jax
import jax.numpy as jnp
from jax.experimental import pallas as pl
from jax.experimental.pallas import tpu as pltpu

_TOP_K = 8
_BLOCK_N = 1024


def _gating_block_kernel(x_ref, w_ref, b_ref, gates_ref, load_ref, *, k_top):
    x = x_ref[...]
    w = w_ref[...]
    e = w.shape[0]
    bn = x.shape[0]
    logits = jax.lax.dot_general(
        w, x, (((1,), (1,)), ((), ())), preferred_element_type=jnp.float32)
    logits = logits + b_ref[...].reshape(e, 1)

    row = jax.lax.broadcasted_iota(jnp.int32, (e, bn), 0)
    work = logits
    m0 = jnp.max(work, axis=0, keepdims=True)
    for t in range(k_top):
        m = m0 if t == 0 else jnp.max(work, axis=0, keepdims=True)
        is_max = work == m
        sel = jnp.min(jnp.where(is_max, row, e), axis=0, keepdims=True)
        work = jnp.where(row == sel, -jnp.inf, work)
    selected = work == -jnp.inf
    ex = jnp.where(selected, jnp.exp(logits - m0), jnp.float32(0.0))
    denom = jnp.sum(ex, axis=0, keepdims=True)
    gates = (ex / denom).T
    gates_ref[...] = gates
    counts = jnp.sum((gates > 0).astype(jnp.int32), axis=0, keepdims=True)

    @pl.when(pl.program_id(1) == 0)
    def _init():
        load_ref[...] = counts[None]

    @pl.when(pl.program_id(1) != 0)
    def _accumulate():
        load_ref[...] += counts[None]


def kernel(x, w_gate, b_gate, w_noise, b_noise):
    del w_noise, b_noise
    n, d = x.shape
    e = w_gate.shape[0]
    bn = min(_BLOCK_N, n)
    nb = n // bn
    cores = 2 if nb % 2 == 0 else 1
    half = nb // cores
    b2 = b_gate.reshape(1, e)

    gates, load3 = pl.pallas_call(
        functools.partial(_gating_block_kernel, k_top=_TOP_K),
        grid=(cores, half),
        in_specs=[
            pl.BlockSpec((bn, d), lambda i, j: (i * half + j, 0)),
            pl.BlockSpec((e, d), lambda i, j: (0, 0)),
            pl.BlockSpec((1, e), lambda i, j: (0, 0)),
        ],
        out_specs=[
            pl.BlockSpec((bn, e), lambda i, j: (i * half + j, 0)),
            pl.BlockSpec((1, 1, e), lambda i, j: (i, 0, 0)),
        ],
        out_shape=[
            jax.ShapeDtypeStruct((n, e), x.dtype),
            jax.ShapeDtypeStruct((cores, 1, e), jnp.int32),
        ],
        compiler_params=pltpu.CompilerParams(
            dimension_semantics=("parallel", "arbitrary")),
    )(x, w_gate, b2)

    load = load3.sum(axis=(0, 1))
    return gates, load
